# baseline (speedup 1.0000x reference)
import functools

import jax
import jax.numpy as jnp
from jax import lax
from jax.experimental import pallas as pl
from jax.experimental.pallas import tpu as pltpu
from jax.experimental.pallas import tpu_sc as plsc

N0, N1, N2 = 10000, 160000, 40000
E0, E1 = 320000, 480000
DF, H, OPH, FH, NF = 128, 64, 64, 16, 8
L, NS, B, NC = 3, 2, 128, 10

NW = 32
G = 1280
SUB = 256
GP = 1024
E0P = 327680
E1P = 491520
BA2P = 163840
BR = 1000

_mesh = plsc.VectorSubcoreMesh(
    core_axis_name="c", subcore_axis_name="s", num_cores=2, num_subcores=16)


def _pad_idx(g, s, ep):
  e = g.shape[0]
  padg = jnp.zeros((ep - e,), jnp.int32)
  pads = jnp.full((ep - e,), 1 << 28, jnp.int32)
  return (jnp.concatenate([g.astype(jnp.int32), padg]),
          jnp.concatenate([s.astype(jnp.int32), pads]))


_WR = 20480
N0P = 20480
N1P = 163840
N2P = 40960
_SPECS = (
    (0, E0P, N0P, 1),
    (1, E1P, N1P, 8),
    (0, E0P, N1P, 8),
    (1, BA2P, N2P, 2),
)


def _segsum_body(y0, y1, g0, s0, g1, s1, gb1, sb1, gb2, sb2,
                 up0p, up1p, ba1p, ba2p,
                 ibuf_g, ibuf_s, cg, cl, lbuf, rows, zbuf, win, sem):
  cid = lax.axis_index("c")
  sid = lax.axis_index("s")
  wid = sid * 2 + cid
  zv = jnp.zeros((16,), jnp.float32)

  def _zb(i, carry):
    r = i // 4
    c = (i % 4) * 16
    zbuf[r, pl.ds(c, 16)] = zv
    return carry
  lax.fori_loop(0, 512, _zb, 0)

  srcs = (y0, y1)
  gaths = (g0, g1, gb1, gb2)
  scats = (s0, s1, sb1, sb2)
  outs = (up0p, up1p, ba1p, ba2p)
  gzv = jnp.zeros((16,), jnp.int32)
  lzv = jnp.full((16,), _WR, jnp.int32)

  stripe = _WR // 16
  for sp in range(4):
    src_id, ep, noutp, nwin = _SPECS[sp]
    src = srcs[src_id]
    chunk = ep // NW
    nb = chunk // G
    for w in range(nwin):
      base = w * _WR
      for q in range(10):
        pltpu.sync_copy(zbuf, win.at[pl.ds(sid * stripe + q * 128, 128)])
      plsc.subcore_barrier()

      def _blk(j, carry):
        ca = pltpu.async_copy(
            gaths[sp].at[pl.ds(wid * chunk + j * G, G)], ibuf_g, sem)
        cb = pltpu.async_copy(
            scats[sp].at[pl.ds(wid * chunk + j * G, G)], ibuf_s, sem)
        ca.wait()
        cb.wait()

        def _scan(k, n):
          gv = ibuf_g[pl.ds(k * 16, 16)]
          sv = ibuf_s[pl.ds(k * 16, 16)]
          loc = sv - base
          msk = (loc >= 0) & (loc < _WR)
          plsc.store_compressed(cg.at[pl.ds(n, 16)], gv, mask=msk)
          plsc.store_compressed(cl.at[pl.ds(n, 16)], loc, mask=msk)
          cnt = plsc.all_reduce_population_count(msk)
          return n + cnt[0]
        n = lax.fori_loop(0, G // 16, _scan, jnp.int32(0))

        for k in range(SUB // 16):
          cg[pl.ds(n + k * 16, 16)] = gzv
          cl[pl.ds(n + k * 16, 16)] = lzv

        nsub = (n + SUB - 1) // SUB

        def _sub(m, c):
          pltpu.async_copy(src.at[cg.at[pl.ds(m * SUB, SUB)]], rows,
                           sem).wait()

          def _cp(k, c2):
            lbuf[pl.ds(k * 16, 16)] = cl[pl.ds(m * SUB + k * 16, 16)]
            return c2
          lax.fori_loop(0, SUB // 16, _cp, 0)
          pltpu.sync_copy(rows, win.at[lbuf], add=True)
          return c
        lax.fori_loop(0, nsub, _sub, 0)
        return carry
      lax.fori_loop(0, nb, _blk, 0)
      plsc.subcore_barrier()
      pltpu.sync_copy(win.at[pl.ds(sid * stripe, stripe)],
                      outs[sp].at[cid, pl.ds(base + sid * stripe, stripe)])


_segsum = pl.kernel(
    _segsum_body,
    out_type=[
        jax.ShapeDtypeStruct((2, N0P, H), jnp.float32),
        jax.ShapeDtypeStruct((2, N1P, H), jnp.float32),
        jax.ShapeDtypeStruct((2, N1P, H), jnp.float32),
        jax.ShapeDtypeStruct((2, N2P, H), jnp.float32),
    ],
    mesh=_mesh,
    scratch_types=[
        pltpu.VMEM((G,), jnp.int32),
        pltpu.VMEM((G,), jnp.int32),
        pltpu.VMEM((G + SUB,), jnp.int32),
        pltpu.VMEM((G + SUB,), jnp.int32),
        pltpu.VMEM((SUB,), jnp.int32),
        pltpu.VMEM((SUB, H), jnp.float32),
        pltpu.VMEM((128, H), jnp.float32),
        pltpu.VMEM_SHARED((_WR + 8, H), jnp.float32),
        pltpu.SemaphoreType.DMA,
    ],
    compiler_params=pltpu.CompilerParams(use_tc_tiling_on_sc=False,
                                         needs_layout_passes=False),
)


_E0NB = E0P // NW // GP


def _ph_body(vpad, dstp, srcp, b0e, emp,
             gdst, gsrc, rows_a, rows_b, b0v, acc, sem):
  cid = lax.axis_index("c")
  sid = lax.axis_index("s")
  wid = sid * 2 + cid
  zv = jnp.zeros((16,), jnp.float32)

  def _za(i, carry):
    acc[i, pl.ds(0, 16)] = zv
    return carry
  lax.fori_loop(0, 136, _za, 0)

  pltpu.sync_copy(b0e, b0v)
  chunk = E0P // NW
  pltpu.sync_copy(dstp.at[pl.ds(wid * chunk, chunk)], gdst)
  pltpu.sync_copy(srcp.at[pl.ds(wid * chunk, chunk)], gsrc)

  def _blk(j, carry):
    ca = pltpu.async_copy(vpad.at[gsrc.at[pl.ds(j * GP, GP)]], rows_a, sem)
    cb = pltpu.async_copy(vpad.at[gdst.at[pl.ds(j * GP, GP)]], rows_b, sem)
    ca.wait()
    cb.wait()

    def _kk(k, c2):
      dv = gdst[pl.ds(j * GP + k * 16, 16)]
      ebv = plsc.load_gather(b0v, [dv])
      for i in range(16):
        e = k * 16 + i
        m = jnp.maximum(rows_a[e, pl.ds(0, 16)], rows_b[e, pl.ds(0, 16)])
        plsc.addupdate(acc.at[ebv[i]], m)
      return c2
    lax.fori_loop(0, GP // 16, _kk, 0)
    return carry
  lax.fori_loop(0, _E0NB, _blk, 0)
  pltpu.sync_copy(acc.at[pl.ds(0, B)], emp.at[wid])


_ph_sc = pl.kernel(
    _ph_body,
    out_type=[jax.ShapeDtypeStruct((NW, B, 16), jnp.float32)],
    mesh=_mesh,
    scratch_types=[
        pltpu.VMEM((_E0NB * GP,), jnp.int32),
        pltpu.VMEM((_E0NB * GP,), jnp.int32),
        pltpu.VMEM((GP, 16), jnp.float32),
        pltpu.VMEM((GP, 16), jnp.float32),
        pltpu.VMEM((N0 + 8,), jnp.int32),
        pltpu.VMEM((136, 16), jnp.float32),
        pltpu.SemaphoreType.DMA,
    ],
    compiler_params=pltpu.CompilerParams(use_tc_tiling_on_sc=False,
                                         needs_layout_passes=False),
)


def _dot(a, b):
  return jnp.dot(a, b, preferred_element_type=jnp.float32)


def _embed_tc(x, w, b):
  n = x.shape[0]

  def body(x_r, w_r, b_r, o_r):
    o_r[...] = _dot(x_r[...], w_r[...]) + b_r[...]

  return pl.pallas_call(
      body,
      grid=(n // BR,),
      in_specs=[
          pl.BlockSpec((BR, DF), lambda i: (i, 0)),
          pl.BlockSpec((DF, H), lambda i: (0, 0)),
          pl.BlockSpec((1, H), lambda i: (0, 0)),
      ],
      out_specs=pl.BlockSpec((BR, H), lambda i: (i, 0)),
      out_shape=jax.ShapeDtypeStruct((n, H), jnp.float32),
  )(x, w, b.reshape(1, H))


def _relu(x):
  return jnp.maximum(x, 0.0)


def _layer_tc(y, a1, a2, w1, b1, w2, b2, wc, bc):
  n = y.shape[0]
  has1 = a1 is not None
  has2 = a2 is not None

  def body(*refs):
    it = iter(refs)
    y_r = next(it)
    a1_r = next(it) if has1 else None
    a2_r = next(it) if has2 else None
    w1_r, b1_r, w2_r, b2_r, wc_r, bc_r, o_r = [next(it) for _ in range(7)]
    yv = y_r[...]
    u = yv + a1_r[0] + a1_r[1] if has1 else yv
    t1 = _relu(_dot(u, w1_r[...]) + b1_r[...])
    v = yv + a2_r[0] + a2_r[1] if has2 else yv
    t2 = _relu(_dot(v, w2_r[...]) + b2_r[...])
    o_r[...] = _relu(_dot(t1 + t2, wc_r[...]) + bc_r[...])

  specs = [pl.BlockSpec((BR, H), lambda i: (i, 0))]
  args = [y]
  for a in (a1, a2):
    if a is not None:
      specs.append(pl.BlockSpec((2, BR, H), lambda i: (0, i, 0)))
      args.append(a)
  for wgt in (w1, b1.reshape(1, H), w2, b2.reshape(1, H), wc, bc.reshape(1, H)):
    specs.append(pl.BlockSpec(wgt.shape, lambda i: tuple(0 for _ in wgt.shape)))
    args.append(wgt)

  return pl.pallas_call(
      body,
      grid=(n // BR,),
      in_specs=specs,
      out_specs=pl.BlockSpec((BR, H), lambda i: (i, 0)),
      out_shape=jax.ShapeDtypeStruct((n, H), jnp.float32),
  )(*args)


def _final_tc(y, z, a1, a2, w1, b1, w2, b2, wc, bc, batch3, emit_cur):
  n = y.shape[0]
  has1 = a1 is not None
  has2 = a2 is not None

  def body(*refs):
    it = iter(refs)
    y_r = next(it)
    z_r = next(it)
    a1_r = next(it) if has1 else None
    a2_r = next(it) if has2 else None
    w1_r, b1_r, w2_r, b2_r, wc_r, bc_r, bat_r = [next(it) for _ in range(7)]
    if emit_cur:
      cur_r = next(it)
    pool_r = next(it)
    yv = y_r[...]
    u = yv + a1_r[0] + a1_r[1] if has1 else yv
    t1 = _relu(_dot(u, w1_r[...]) + b1_r[...])
    v = yv + a2_r[0] + a2_r[1] if has2 else yv
    t2 = _relu(_dot(v, w2_r[...]) + b2_r[...])
    cur = z_r[...] + _relu(_dot(t1 + t2, wc_r[...]) + bc_r[...])
    if emit_cur:
      cur_r[...] = cur
    i = pl.program_id(0)

    @pl.when(i == 0)
    def _():
      pool_r[...] = jnp.zeros((B, H), jnp.float32)

    bvec = bat_r[0, 0, :]
    oh = (bvec[:, None] == lax.broadcasted_iota(jnp.int32, (BR, B), 1)
          ).astype(jnp.float32)
    pool_r[...] += lax.dot_general(
        oh, cur, dimension_numbers=(((0,), (0,)), ((), ())),
        preferred_element_type=jnp.float32)

  specs = [pl.BlockSpec((BR, H), lambda i: (i, 0)),
           pl.BlockSpec((BR, H), lambda i: (i, 0))]
  args = [y, z]
  for a in (a1, a2):
    if a is not None:
      specs.append(pl.BlockSpec((2, BR, H), lambda i: (0, i, 0)))
      args.append(a)
  for wgt in (w1, b1.reshape(1, H), w2, b2.reshape(1, H), wc, bc.reshape(1, H)):
    specs.append(pl.BlockSpec(wgt.shape, lambda i: tuple(0 for _ in wgt.shape)))
    args.append(wgt)
  specs.append(pl.BlockSpec((1, 1, BR), lambda i: (i, 0, 0)))
  args.append(batch3)

  out_specs = []
  out_shape = []
  if emit_cur:
    out_specs.append(pl.BlockSpec((BR, H), lambda i: (i, 0)))
    out_shape.append(jax.ShapeDtypeStruct((n, H), jnp.float32))
  out_specs.append(pl.BlockSpec((B, H), lambda i: (0, 0)))
  out_shape.append(jax.ShapeDtypeStruct((B, H), jnp.float32))

  return pl.pallas_call(
      body,
      grid=(n // BR,),
      in_specs=specs,
      out_specs=out_specs,
      out_shape=out_shape,
  )(*args)


def _v_tc(x, w1, b1, w2, b2, batch3):
  n = x.shape[0]

  def body(x_r, w1_r, b1_r, w2_r, b2_r, bat_r, vp_r, nm_r):
    h = _relu(_dot(x_r[...], w1_r[...]) + b1_r[...])
    v = jax.nn.sigmoid(_dot(h, w2_r[...]) + b2_r[...])
    vp = jnp.concatenate(
        [v, jnp.ones((BR, 1), jnp.float32), jnp.zeros((BR, 7), jnp.float32)],
        axis=1)
    vp_r[...] = vp
    i = pl.program_id(0)

    @pl.when(i == 0)
    def _():
      nm_r[...] = jnp.zeros((B, 16), jnp.float32)

    bvec = bat_r[0, 0, :]
    oh = (bvec[:, None] == lax.broadcasted_iota(jnp.int32, (BR, B), 1)
          ).astype(jnp.float32)
    nm_r[...] += lax.dot_general(
        oh, vp, dimension_numbers=(((0,), (0,)), ((), ())),
        preferred_element_type=jnp.float32)

  return pl.pallas_call(
      body,
      grid=(n // BR,),
      in_specs=[
          pl.BlockSpec((BR, H), lambda i: (i, 0)),
          pl.BlockSpec((H, FH), lambda i: (0, 0)),
          pl.BlockSpec((1, FH), lambda i: (0, 0)),
          pl.BlockSpec((FH, NF), lambda i: (0, 0)),
          pl.BlockSpec((1, NF), lambda i: (0, 0)),
          pl.BlockSpec((1, 1, BR), lambda i: (i, 0, 0)),
      ],
      out_specs=[
          pl.BlockSpec((BR, 16), lambda i: (i, 0)),
          pl.BlockSpec((B, 16), lambda i: (0, 0)),
      ],
      out_shape=[
          jax.ShapeDtypeStruct((n, 16), jnp.float32),
          jax.ShapeDtypeStruct((B, 16), jnp.float32),
      ],
  )(x, w1, b1.reshape(1, FH), w2, b2.reshape(1, NF), batch3)


def _head_tc(pools, nms, emps, phws, phbs, l1ws, l1bs, l2w, l2b):

  def body(p0_r, p1_r, p2_r, nm0_r, nm1_r, e0_r, e1_r,
           pw0_r, pb0_r, pw1_r, pb1_r,
           lw0_r, lb0_r, lw1_r, lb1_r, lw2_r, lb2_r, l2w_r, l2b_r, o_r):
    def ph(nm_r, emp_r, pw_r, pb_r):
      em_t = jnp.sum(emp_r[...], axis=0)
      ce = jnp.clip(em_t[:, 8:9], 1.0, None)
      emv = em_t[:, 0:8] / ce
      nm_t = nm_r[...]
      c0 = jnp.clip(nm_t[:, 8:9], 1.0, None)
      nmv = nm_t[:, 0:8] / c0
      feat = jnp.concatenate([nmv, emv], axis=1)
      return _relu(_dot(feat, pw_r[...]) + pb_r[...])

    ph0 = ph(nm0_r, e0_r, pw0_r, pb0_r)
    ph1 = ph(nm1_r, e1_r, pw1_r, pb1_r)
    phe = 0.5 * (ph0 + ph1)
    x = (_relu(_dot(p0_r[...], lw0_r[...]) + lb0_r[...]) +
         _relu(_dot(p1_r[...], lw1_r[...]) + lb1_r[...]) +
         _relu(_dot(p2_r[...], lw2_r[...]) + lb2_r[...]))
    o_r[...] = _dot(jnp.concatenate([x, phe], axis=1), l2w_r[...]) + l2b_r[...]

  args = [pools[0], pools[1], pools[2], nms[0], nms[1], emps[0], emps[1],
          phws[0], phbs[0].reshape(1, OPH), phws[1], phbs[1].reshape(1, OPH),
          l1ws[0], l1bs[0].reshape(1, 2 * H), l1ws[1], l1bs[1].reshape(1, 2 * H),
          l1ws[2], l1bs[2].reshape(1, 2 * H), l2w, l2b.reshape(1, 16)]
  return pl.pallas_call(
      body,
      out_shape=jax.ShapeDtypeStruct((B, 16), jnp.float32),
  )(*args)


def kernel(x0, x1, x2, up_index0, up_index1, bdry1_src, bdry1_dst,
           bdry2_src, bdry2_dst, batch0, batch1, batch2, params):
  p = params
  g0, s0 = _pad_idx(up_index0[1], up_index0[0], E0P)
  g1, s1 = _pad_idx(up_index1[1], up_index1[0], E1P)
  gb1, sb1 = _pad_idx(bdry1_src, bdry1_dst, E0P)
  gb2, sb2 = _pad_idx(bdry2_src, bdry2_dst, BA2P)

  z0 = _embed_tc(x0, p["embed_W"][0], p["embed_b"][0])
  z1 = _embed_tc(x1, p["embed_W"][1], p["embed_b"][1])
  z2 = _embed_tc(x2, p["embed_W"][2], p["embed_b"][2])

  batch3 = [batch0.reshape(-1, 1, BR).astype(jnp.int32),
            batch1.reshape(-1, 1, BR).astype(jnp.int32),
            batch2.reshape(-1, 1, BR).astype(jnp.int32)]

  y = [z0, z1, z2]
  zs = [z0, z1, z2]
  pools = [None, None, None]
  cur0 = None
  for l in range(L):
    if l == 0:
      w1l = [p["Wup0"][d, 1:, :] for d in range(3)]
      b1l = [p["bup0"][d] for d in range(3)]
      w2l = [p["Wb0"][d, 1:, :] for d in range(3)]
      b2l = [p["bb0"][d] for d in range(3)]
      wcl = [p["Wc0"][d] for d in range(3)]
      bcl = [p["bc0"][d] for d in range(3)]
    else:
      w1l = [p["WupR"][l - 1, d] for d in range(3)]
      b1l = [p["bupR"][l - 1, d] for d in range(3)]
      w2l = [p["WbR"][l - 1, d] for d in range(3)]
      b2l = [p["bbR"][l - 1, d] for d in range(3)]
      wcl = [p["WcR"][l - 1, d] for d in range(3)]
      bcl = [p["bcR"][l - 1, d] for d in range(3)]

    up0p, up1p, ba1p, ba2p = _segsum(y[0], y[1], g0, s0, g1, s1,
                                     gb1, sb1, gb2, sb2)
    a1s = [up0p, up1p, None]
    a2s = [None, ba1p, ba2p]
    if l < L - 1:
      y = [_layer_tc(y[d], a1s[d], a2s[d], w1l[d], b1l[d], w2l[d], b2l[d],
                     wcl[d], bcl[d]) for d in range(3)]
    else:
      cur0, pools[0] = _final_tc(y[0], zs[0], a1s[0], a2s[0], w1l[0], b1l[0],
                                 w2l[0], b2l[0], wcl[0], bcl[0], batch3[0],
                                 emit_cur=True)
      pools[1], = _final_tc(y[1], zs[1], a1s[1], a2s[1], w1l[1], b1l[1],
                            w2l[1], b2l[1], wcl[1], bcl[1], batch3[1],
                            emit_cur=False)
      pools[2], = _final_tc(y[2], zs[2], a1s[2], a2s[2], w1l[2], b1l[2],
                            w2l[2], b2l[2], wcl[2], bcl[2], batch3[2],
                            emit_cur=False)

  vpad0, nm0 = _v_tc(z0, p["filW1"][0], p["filb1"][0], p["filW2"][0],
                     p["filb2"][0], batch3[0])
  vpad1, nm1 = _v_tc(cur0, p["filW1"][1], p["filb1"][1], p["filW2"][1],
                     p["filb2"][1], batch3[0])

  padrows = jnp.zeros((8, 16), jnp.float32)
  b0e = jnp.concatenate([batch0.astype(jnp.int32),
                         jnp.full((8,), B, jnp.int32)])
  dpad = jnp.full((E0P - E0,), N0, jnp.int32)
  dstp = jnp.concatenate([up_index0[0].astype(jnp.int32), dpad])
  srcp = jnp.concatenate([up_index0[1].astype(jnp.int32), dpad])

  emp0, = _ph_sc(jnp.concatenate([vpad0, padrows]), dstp, srcp, b0e)
  emp1, = _ph_sc(jnp.concatenate([vpad1, padrows]), dstp, srcp, b0e)

  l2w = jnp.concatenate([p["lin2W"], jnp.zeros((2 * H + OPH, 6), jnp.float32)],
                        axis=1)
  l2b = jnp.concatenate([p["lin2b"], jnp.zeros((6,), jnp.float32)])
  out16 = _head_tc(pools, [nm0, nm1], [emp0, emp1],
                   [p["phW"][0], p["phW"][1]], [p["phb"][0], p["phb"][1]],
                   [p["lin1W"][d] for d in range(3)],
                   [p["lin1b"][d] for d in range(3)], l2w, l2b)
  return out16[:, :NC]


# speedup vs baseline: 1.8030x; 1.8030x over previous
import functools

import jax
import jax.numpy as jnp
from jax import lax
from jax.experimental import pallas as pl
from jax.experimental.pallas import tpu as pltpu
from jax.experimental.pallas import tpu_sc as plsc

N0, N1, N2 = 10000, 160000, 40000
E0, E1 = 320000, 480000
DF, H, OPH, FH, NF = 128, 64, 64, 16, 8
L, NS, B, NC = 3, 2, 128, 10

NW = 32
G = 256
GP = 1024
E0P = 327680
E1P = 491520
BA2P = 163840
BR = 1000

_mesh = plsc.VectorSubcoreMesh(
    core_axis_name="c", subcore_axis_name="s", num_cores=2, num_subcores=16)


def _pad_idx(g, s, ep):
  e = g.shape[0]
  padg = jnp.zeros((ep - e,), jnp.int32)
  pads = jnp.full((ep - e,), 1 << 28, jnp.int32)
  return (jnp.concatenate([g.astype(jnp.int32), padg]),
          jnp.concatenate([s.astype(jnp.int32), pads]))


_WR = 20480
N0P = 20480
N1P = 163840
N2P = 40960
_SPECS = (
    (0, E0P, N0P, 1),
    (1, E1P, N1P, 8),
    (0, E0P, N1P, 8),
    (1, BA2P, N2P, 2),
)


def _segsum_body(y0, y1, g0, s0, g1, s1, gb1, sb1, gb2, sb2,
                 up0p, up1p, ba1p, ba2p,
                 ig0, is0, ig1, is1, lb0, lb1, rows0, rows1,
                 zbuf, win, semi, sg0, sg1):
  cid = lax.axis_index("c")
  sid = lax.axis_index("s")
  wid = sid * 2 + cid
  zv = jnp.zeros((16,), jnp.float32)

  def _zb(i, carry):
    r = i // 4
    c = (i % 4) * 16
    zbuf[r, pl.ds(c, 16)] = zv
    return carry
  lax.fori_loop(0, 256, _zb, 0)

  srcs = (y0, y1)
  gaths = (g0, g1, gb1, gb2)
  scats = (s0, s1, sb1, sb2)
  outs = (up0p, up1p, ba1p, ba2p)
  igs = (ig0, ig1)
  iss = (is0, is1)
  lbs = (lb0, lb1)
  rws = (rows0, rows1)
  sgs = (sg0, sg1)

  stripe = _WR // 16
  for sp in range(4):
    src_id, ep, noutp, nwin = _SPECS[sp]
    src = srcs[src_id]
    chunk = ep // NW
    nb = chunk // G
    gref = gaths[sp]
    sref = scats[sp]

    def _stage(j, b, base):
      ca = pltpu.async_copy(gref.at[pl.ds(wid * chunk + j * G, G)],
                            igs[b], semi)
      cb = pltpu.async_copy(sref.at[pl.ds(wid * chunk + j * G, G)],
                            iss[b], semi)
      ca.wait()
      cb.wait()

      def _kk(k, c2):
        v = iss[b][pl.ds(k * 16, 16)]
        loc = v - base
        msk = (loc >= 0) & (loc < _WR)
        lbs[b][pl.ds(k * 16, 16)] = jnp.where(msk, loc, _WR)
        return c2
      lax.fori_loop(0, G // 16, _kk, 0)
      pltpu.async_copy(src.at[igs[b]], rws[b], sgs[b])

    for w in range(nwin):
      base = w * _WR
      for q in range(20):
        pltpu.sync_copy(zbuf, win.at[pl.ds(sid * stripe + q * 64, 64)])
      plsc.subcore_barrier()

      _stage(0, 0, base)

      def _blk(jj, carry):
        for par in range(2):
          j = jj * 2 + par
          b = par
          _stage(j + 1, 1 - b, base)
          pltpu.make_async_copy(src.at[igs[b]], rws[b], sgs[b]).wait()
          pltpu.sync_copy(rws[b], win.at[lbs[b]], add=True)
        return carry
      lax.fori_loop(0, (nb - 2) // 2, _blk, 0)

      for j in (nb - 2, nb - 1):
        b = j % 2
        if j < nb - 1:
          _stage(j + 1, 1 - b, base)
        pltpu.make_async_copy(src.at[igs[b]], rws[b], sgs[b]).wait()
        pltpu.sync_copy(rws[b], win.at[lbs[b]], add=True)
      plsc.subcore_barrier()
      pltpu.sync_copy(win.at[pl.ds(sid * stripe, stripe)],
                      outs[sp].at[cid, pl.ds(base + sid * stripe, stripe)])


_segsum = pl.kernel(
    _segsum_body,
    out_type=[
        jax.ShapeDtypeStruct((2, N0P, H), jnp.float32),
        jax.ShapeDtypeStruct((2, N1P, H), jnp.float32),
        jax.ShapeDtypeStruct((2, N1P, H), jnp.float32),
        jax.ShapeDtypeStruct((2, N2P, H), jnp.float32),
    ],
    mesh=_mesh,
    scratch_types=[
        pltpu.VMEM((G,), jnp.int32),
        pltpu.VMEM((G,), jnp.int32),
        pltpu.VMEM((G,), jnp.int32),
        pltpu.VMEM((G,), jnp.int32),
        pltpu.VMEM((G,), jnp.int32),
        pltpu.VMEM((G,), jnp.int32),
        pltpu.VMEM((G, H), jnp.float32),
        pltpu.VMEM((G, H), jnp.float32),
        pltpu.VMEM((64, H), jnp.float32),
        pltpu.VMEM_SHARED((_WR + 8, H), jnp.float32),
        pltpu.SemaphoreType.DMA,
        pltpu.SemaphoreType.DMA,
        pltpu.SemaphoreType.DMA,
    ],
    compiler_params=pltpu.CompilerParams(use_tc_tiling_on_sc=False,
                                         needs_layout_passes=False),
)


_E0NB = E0P // NW // GP


def _ph_body(vpad, dstp, srcp, b0e, emp,
             gdst, gsrc, rows_a, rows_b, b0v, acc, sem):
  cid = lax.axis_index("c")
  sid = lax.axis_index("s")
  wid = sid * 2 + cid
  zv = jnp.zeros((16,), jnp.float32)

  def _za(i, carry):
    acc[i, pl.ds(0, 16)] = zv
    return carry
  lax.fori_loop(0, 136, _za, 0)

  pltpu.sync_copy(b0e, b0v)
  chunk = E0P // NW
  pltpu.sync_copy(dstp.at[pl.ds(wid * chunk, chunk)], gdst)
  pltpu.sync_copy(srcp.at[pl.ds(wid * chunk, chunk)], gsrc)

  def _blk(j, carry):
    ca = pltpu.async_copy(vpad.at[gsrc.at[pl.ds(j * GP, GP)]], rows_a, sem)
    cb = pltpu.async_copy(vpad.at[gdst.at[pl.ds(j * GP, GP)]], rows_b, sem)
    ca.wait()
    cb.wait()

    def _kk(k, c2):
      dv = gdst[pl.ds(j * GP + k * 16, 16)]
      ebv = plsc.load_gather(b0v, [dv])
      for i in range(16):
        e = k * 16 + i
        m = jnp.maximum(rows_a[e, pl.ds(0, 16)], rows_b[e, pl.ds(0, 16)])
        plsc.addupdate(acc.at[ebv[i]], m)
      return c2
    lax.fori_loop(0, GP // 16, _kk, 0)
    return carry
  lax.fori_loop(0, _E0NB, _blk, 0)
  pltpu.sync_copy(acc.at[pl.ds(0, B)], emp.at[wid])


_ph_sc = pl.kernel(
    _ph_body,
    out_type=[jax.ShapeDtypeStruct((NW, B, 16), jnp.float32)],
    mesh=_mesh,
    scratch_types=[
        pltpu.VMEM((_E0NB * GP,), jnp.int32),
        pltpu.VMEM((_E0NB * GP,), jnp.int32),
        pltpu.VMEM((GP, 16), jnp.float32),
        pltpu.VMEM((GP, 16), jnp.float32),
        pltpu.VMEM((N0 + 8,), jnp.int32),
        pltpu.VMEM((136, 16), jnp.float32),
        pltpu.SemaphoreType.DMA,
    ],
    compiler_params=pltpu.CompilerParams(use_tc_tiling_on_sc=False,
                                         needs_layout_passes=False),
)


def _dot(a, b):
  return jnp.dot(a, b, preferred_element_type=jnp.float32)


def _embed_tc(x, w, b):
  n = x.shape[0]

  def body(x_r, w_r, b_r, o_r):
    o_r[...] = _dot(x_r[...], w_r[...]) + b_r[...]

  return pl.pallas_call(
      body,
      grid=(n // BR,),
      in_specs=[
          pl.BlockSpec((BR, DF), lambda i: (i, 0)),
          pl.BlockSpec((DF, H), lambda i: (0, 0)),
          pl.BlockSpec((1, H), lambda i: (0, 0)),
      ],
      out_specs=pl.BlockSpec((BR, H), lambda i: (i, 0)),
      out_shape=jax.ShapeDtypeStruct((n, H), jnp.float32),
  )(x, w, b.reshape(1, H))


def _relu(x):
  return jnp.maximum(x, 0.0)


def _layer_tc(y, a1, a2, w1, b1, w2, b2, wc, bc):
  n = y.shape[0]
  has1 = a1 is not None
  has2 = a2 is not None

  def body(*refs):
    it = iter(refs)
    y_r = next(it)
    a1_r = next(it) if has1 else None
    a2_r = next(it) if has2 else None
    w1_r, b1_r, w2_r, b2_r, wc_r, bc_r, o_r = [next(it) for _ in range(7)]
    yv = y_r[...]
    u = yv + a1_r[0] + a1_r[1] if has1 else yv
    t1 = _relu(_dot(u, w1_r[...]) + b1_r[...])
    v = yv + a2_r[0] + a2_r[1] if has2 else yv
    t2 = _relu(_dot(v, w2_r[...]) + b2_r[...])
    o_r[...] = _relu(_dot(t1 + t2, wc_r[...]) + bc_r[...])

  specs = [pl.BlockSpec((BR, H), lambda i: (i, 0))]
  args = [y]
  for a in (a1, a2):
    if a is not None:
      specs.append(pl.BlockSpec((2, BR, H), lambda i: (0, i, 0)))
      args.append(a)
  for wgt in (w1, b1.reshape(1, H), w2, b2.reshape(1, H), wc, bc.reshape(1, H)):
    specs.append(pl.BlockSpec(wgt.shape, lambda i: tuple(0 for _ in wgt.shape)))
    args.append(wgt)

  return pl.pallas_call(
      body,
      grid=(n // BR,),
      in_specs=specs,
      out_specs=pl.BlockSpec((BR, H), lambda i: (i, 0)),
      out_shape=jax.ShapeDtypeStruct((n, H), jnp.float32),
  )(*args)


def _final_tc(y, z, a1, a2, w1, b1, w2, b2, wc, bc, batch3, emit_cur):
  n = y.shape[0]
  has1 = a1 is not None
  has2 = a2 is not None

  def body(*refs):
    it = iter(refs)
    y_r = next(it)
    z_r = next(it)
    a1_r = next(it) if has1 else None
    a2_r = next(it) if has2 else None
    w1_r, b1_r, w2_r, b2_r, wc_r, bc_r, bat_r = [next(it) for _ in range(7)]
    if emit_cur:
      cur_r = next(it)
    pool_r = next(it)
    yv = y_r[...]
    u = yv + a1_r[0] + a1_r[1] if has1 else yv
    t1 = _relu(_dot(u, w1_r[...]) + b1_r[...])
    v = yv + a2_r[0] + a2_r[1] if has2 else yv
    t2 = _relu(_dot(v, w2_r[...]) + b2_r[...])
    cur = z_r[...] + _relu(_dot(t1 + t2, wc_r[...]) + bc_r[...])
    if emit_cur:
      cur_r[...] = cur
    i = pl.program_id(0)

    @pl.when(i == 0)
    def _():
      pool_r[...] = jnp.zeros((B, H), jnp.float32)

    bvec = bat_r[0, 0, :]
    oh = (bvec[:, None] == lax.broadcasted_iota(jnp.int32, (BR, B), 1)
          ).astype(jnp.float32)
    pool_r[...] += lax.dot_general(
        oh, cur, dimension_numbers=(((0,), (0,)), ((), ())),
        preferred_element_type=jnp.float32)

  specs = [pl.BlockSpec((BR, H), lambda i: (i, 0)),
           pl.BlockSpec((BR, H), lambda i: (i, 0))]
  args = [y, z]
  for a in (a1, a2):
    if a is not None:
      specs.append(pl.BlockSpec((2, BR, H), lambda i: (0, i, 0)))
      args.append(a)
  for wgt in (w1, b1.reshape(1, H), w2, b2.reshape(1, H), wc, bc.reshape(1, H)):
    specs.append(pl.BlockSpec(wgt.shape, lambda i: tuple(0 for _ in wgt.shape)))
    args.append(wgt)
  specs.append(pl.BlockSpec((1, 1, BR), lambda i: (i, 0, 0)))
  args.append(batch3)

  out_specs = []
  out_shape = []
  if emit_cur:
    out_specs.append(pl.BlockSpec((BR, H), lambda i: (i, 0)))
    out_shape.append(jax.ShapeDtypeStruct((n, H), jnp.float32))
  out_specs.append(pl.BlockSpec((B, H), lambda i: (0, 0)))
  out_shape.append(jax.ShapeDtypeStruct((B, H), jnp.float32))

  return pl.pallas_call(
      body,
      grid=(n // BR,),
      in_specs=specs,
      out_specs=out_specs,
      out_shape=out_shape,
  )(*args)


def _v_tc(x, w1, b1, w2, b2, batch3):
  n = x.shape[0]

  def body(x_r, w1_r, b1_r, w2_r, b2_r, bat_r, vp_r, nm_r):
    h = _relu(_dot(x_r[...], w1_r[...]) + b1_r[...])
    v = jax.nn.sigmoid(_dot(h, w2_r[...]) + b2_r[...])
    vp = jnp.concatenate(
        [v, jnp.ones((BR, 1), jnp.float32), jnp.zeros((BR, 7), jnp.float32)],
        axis=1)
    vp_r[...] = vp
    i = pl.program_id(0)

    @pl.when(i == 0)
    def _():
      nm_r[...] = jnp.zeros((B, 16), jnp.float32)

    bvec = bat_r[0, 0, :]
    oh = (bvec[:, None] == lax.broadcasted_iota(jnp.int32, (BR, B), 1)
          ).astype(jnp.float32)
    nm_r[...] += lax.dot_general(
        oh, vp, dimension_numbers=(((0,), (0,)), ((), ())),
        preferred_element_type=jnp.float32)

  return pl.pallas_call(
      body,
      grid=(n // BR,),
      in_specs=[
          pl.BlockSpec((BR, H), lambda i: (i, 0)),
          pl.BlockSpec((H, FH), lambda i: (0, 0)),
          pl.BlockSpec((1, FH), lambda i: (0, 0)),
          pl.BlockSpec((FH, NF), lambda i: (0, 0)),
          pl.BlockSpec((1, NF), lambda i: (0, 0)),
          pl.BlockSpec((1, 1, BR), lambda i: (i, 0, 0)),
      ],
      out_specs=[
          pl.BlockSpec((BR, 16), lambda i: (i, 0)),
          pl.BlockSpec((B, 16), lambda i: (0, 0)),
      ],
      out_shape=[
          jax.ShapeDtypeStruct((n, 16), jnp.float32),
          jax.ShapeDtypeStruct((B, 16), jnp.float32),
      ],
  )(x, w1, b1.reshape(1, FH), w2, b2.reshape(1, NF), batch3)


def _head_tc(pools, nms, emps, phws, phbs, l1ws, l1bs, l2w, l2b):

  def body(p0_r, p1_r, p2_r, nm0_r, nm1_r, e0_r, e1_r,
           pw0_r, pb0_r, pw1_r, pb1_r,
           lw0_r, lb0_r, lw1_r, lb1_r, lw2_r, lb2_r, l2w_r, l2b_r, o_r):
    def ph(nm_r, emp_r, pw_r, pb_r):
      em_t = jnp.sum(emp_r[...], axis=0)
      ce = jnp.clip(em_t[:, 8:9], 1.0, None)
      emv = em_t[:, 0:8] / ce
      nm_t = nm_r[...]
      c0 = jnp.clip(nm_t[:, 8:9], 1.0, None)
      nmv = nm_t[:, 0:8] / c0
      feat = jnp.concatenate([nmv, emv], axis=1)
      return _relu(_dot(feat, pw_r[...]) + pb_r[...])

    ph0 = ph(nm0_r, e0_r, pw0_r, pb0_r)
    ph1 = ph(nm1_r, e1_r, pw1_r, pb1_r)
    phe = 0.5 * (ph0 + ph1)
    x = (_relu(_dot(p0_r[...], lw0_r[...]) + lb0_r[...]) +
         _relu(_dot(p1_r[...], lw1_r[...]) + lb1_r[...]) +
         _relu(_dot(p2_r[...], lw2_r[...]) + lb2_r[...]))
    o_r[...] = _dot(jnp.concatenate([x, phe], axis=1), l2w_r[...]) + l2b_r[...]

  args = [pools[0], pools[1], pools[2], nms[0], nms[1], emps[0], emps[1],
          phws[0], phbs[0].reshape(1, OPH), phws[1], phbs[1].reshape(1, OPH),
          l1ws[0], l1bs[0].reshape(1, 2 * H), l1ws[1], l1bs[1].reshape(1, 2 * H),
          l1ws[2], l1bs[2].reshape(1, 2 * H), l2w, l2b.reshape(1, 16)]
  return pl.pallas_call(
      body,
      out_shape=jax.ShapeDtypeStruct((B, 16), jnp.float32),
  )(*args)


def kernel(x0, x1, x2, up_index0, up_index1, bdry1_src, bdry1_dst,
           bdry2_src, bdry2_dst, batch0, batch1, batch2, params):
  p = params
  g0, s0 = _pad_idx(up_index0[1], up_index0[0], E0P)
  g1, s1 = _pad_idx(up_index1[1], up_index1[0], E1P)
  gb1, sb1 = _pad_idx(bdry1_src, bdry1_dst, E0P)
  gb2, sb2 = _pad_idx(bdry2_src, bdry2_dst, BA2P)

  z0 = _embed_tc(x0, p["embed_W"][0], p["embed_b"][0])
  z1 = _embed_tc(x1, p["embed_W"][1], p["embed_b"][1])
  z2 = _embed_tc(x2, p["embed_W"][2], p["embed_b"][2])

  batch3 = [batch0.reshape(-1, 1, BR).astype(jnp.int32),
            batch1.reshape(-1, 1, BR).astype(jnp.int32),
            batch2.reshape(-1, 1, BR).astype(jnp.int32)]

  y = [z0, z1, z2]
  zs = [z0, z1, z2]
  pools = [None, None, None]
  cur0 = None
  for l in range(L):
    if l == 0:
      w1l = [p["Wup0"][d, 1:, :] for d in range(3)]
      b1l = [p["bup0"][d] for d in range(3)]
      w2l = [p["Wb0"][d, 1:, :] for d in range(3)]
      b2l = [p["bb0"][d] for d in range(3)]
      wcl = [p["Wc0"][d] for d in range(3)]
      bcl = [p["bc0"][d] for d in range(3)]
    else:
      w1l = [p["WupR"][l - 1, d] for d in range(3)]
      b1l = [p["bupR"][l - 1, d] for d in range(3)]
      w2l = [p["WbR"][l - 1, d] for d in range(3)]
      b2l = [p["bbR"][l - 1, d] for d in range(3)]
      wcl = [p["WcR"][l - 1, d] for d in range(3)]
      bcl = [p["bcR"][l - 1, d] for d in range(3)]

    up0p, up1p, ba1p, ba2p = _segsum(y[0], y[1], g0, s0, g1, s1,
                                     gb1, sb1, gb2, sb2)
    a1s = [up0p, up1p, None]
    a2s = [None, ba1p, ba2p]
    if l < L - 1:
      y = [_layer_tc(y[d], a1s[d], a2s[d], w1l[d], b1l[d], w2l[d], b2l[d],
                     wcl[d], bcl[d]) for d in range(3)]
    else:
      cur0, pools[0] = _final_tc(y[0], zs[0], a1s[0], a2s[0], w1l[0], b1l[0],
                                 w2l[0], b2l[0], wcl[0], bcl[0], batch3[0],
                                 emit_cur=True)
      pools[1], = _final_tc(y[1], zs[1], a1s[1], a2s[1], w1l[1], b1l[1],
                            w2l[1], b2l[1], wcl[1], bcl[1], batch3[1],
                            emit_cur=False)
      pools[2], = _final_tc(y[2], zs[2], a1s[2], a2s[2], w1l[2], b1l[2],
                            w2l[2], b2l[2], wcl[2], bcl[2], batch3[2],
                            emit_cur=False)

  vpad0, nm0 = _v_tc(z0, p["filW1"][0], p["filb1"][0], p["filW2"][0],
                     p["filb2"][0], batch3[0])
  vpad1, nm1 = _v_tc(cur0, p["filW1"][1], p["filb1"][1], p["filW2"][1],
                     p["filb2"][1], batch3[0])

  padrows = jnp.zeros((8, 16), jnp.float32)
  b0e = jnp.concatenate([batch0.astype(jnp.int32),
                         jnp.full((8,), B, jnp.int32)])
  dpad = jnp.full((E0P - E0,), N0, jnp.int32)
  dstp = jnp.concatenate([up_index0[0].astype(jnp.int32), dpad])
  srcp = jnp.concatenate([up_index0[1].astype(jnp.int32), dpad])

  emp0, = _ph_sc(jnp.concatenate([vpad0, padrows]), dstp, srcp, b0e)
  emp1, = _ph_sc(jnp.concatenate([vpad1, padrows]), dstp, srcp, b0e)

  l2w = jnp.concatenate([p["lin2W"], jnp.zeros((2 * H + OPH, 6), jnp.float32)],
                        axis=1)
  l2b = jnp.concatenate([p["lin2b"], jnp.zeros((6,), jnp.float32)])
  out16 = _head_tc(pools, [nm0, nm1], [emp0, emp1],
                   [p["phW"][0], p["phW"][1]], [p["phb"][0], p["phb"][1]],
                   [p["lin1W"][d] for d in range(3)],
                   [p["lin1b"][d] for d in range(3)], l2w, l2b)
  return out16[:, :NC]


# speedup vs baseline: 4.0236x; 2.2317x over previous
import functools

import jax
import jax.numpy as jnp
from jax import lax
from jax.experimental import pallas as pl
from jax.experimental.pallas import tpu as pltpu
from jax.experimental.pallas import tpu_sc as plsc

N0, N1, N2 = 10000, 160000, 40000
E0, E1 = 320000, 480000
DF, H, OPH, FH, NF = 128, 64, 64, 16, 8
L, NS, B, NC = 3, 2, 128, 10

NW = 32
G = 256
GP = 1024
E0P = 327680
E1P = 491520
BA2P = 163840
BR = 1000

_mesh = plsc.VectorSubcoreMesh(
    core_axis_name="c", subcore_axis_name="s", num_cores=2, num_subcores=16)


def _pad_idx(g, s, ep):
  e = g.shape[0]
  padg = jnp.zeros((ep - e,), jnp.int32)
  pads = jnp.full((ep - e,), 1 << 28, jnp.int32)
  return (jnp.concatenate([g.astype(jnp.int32), padg]),
          jnp.concatenate([s.astype(jnp.int32), pads]))


_WR = 20480
N0P = 20480
N1P = 163840
N2P = 40960
_SPECS = (
    (0, E0P, N0P, 1),
    (1, E1P, N1P, 8),
    (0, E0P, N1P, 8),
    (1, BA2P, N2P, 2),
)
_SWBASE = (0, 1, 9, 17)


def _bucket_body(g0, s0, g1, s1, gb1, sb1, gb2, sb2,
                 bg0, bl0, bg1, bl1, bgb1, blb1, bgb2, blb2, cnts,
                 cig, cis, stg, stl, cvbuf, sem):
  cid = lax.axis_index("c")
  sid = lax.axis_index("s")
  wid = sid * 2 + cid
  gzv = jnp.zeros((16,), jnp.int32)
  lzv = jnp.full((16,), _WR, jnp.int32)
  lanes = lax.iota(jnp.int32, 16)
  cvbuf[pl.ds(0, 16)] = gzv
  cvbuf[pl.ds(16, 16)] = gzv

  gaths = (g0, g1, gb1, gb2)
  scats = (s0, s1, sb1, sb2)
  bgs = (bg0, bg1, bgb1, bgb2)
  bls = (bl0, bl1, blb1, blb2)

  for sp in range(4):
    _, ep, _, nwin = _SPECS[sp]
    chunk = ep // NW
    ca = pltpu.async_copy(gaths[sp].at[pl.ds(wid * chunk, chunk)],
                          cig.at[pl.ds(0, chunk)], sem)
    cb = pltpu.async_copy(scats[sp].at[pl.ds(wid * chunk, chunk)],
                          cis.at[pl.ds(0, chunk)], sem)
    ca.wait()
    cb.wait()
    for w in range(nwin):
      base = w * _WR
      sw = _SWBASE[sp] + w

      def _scan(k, n):
        gv = cig[pl.ds(k * 16, 16)]
        sv = cis[pl.ds(k * 16, 16)]
        loc = sv - base
        msk = (loc >= 0) & (loc < _WR)
        plsc.store_compressed(stg.at[pl.ds(n, 16)], gv, mask=msk)
        plsc.store_compressed(stl.at[pl.ds(n, 16)], loc, mask=msk)
        return n + plsc.all_reduce_population_count(msk)[0]
      n = lax.fori_loop(0, chunk // 16, _scan, jnp.int32(0))

      for k in range(G // 16):
        stg[pl.ds(n + k * 16, 16)] = gzv
        stl[pl.ds(n + k * 16, 16)] = lzv
      nblk = jnp.maximum((n + G - 1) // G, 1)

      def _wr(m, c):
        pltpu.sync_copy(stg.at[pl.ds(m * G, G)],
                        bgs[sp].at[wid, w, pl.ds(m * G, G)])
        pltpu.sync_copy(stl.at[pl.ds(m * G, G)],
                        bls[sp].at[wid, w, pl.ds(m * G, G)])
        return c
      lax.fori_loop(0, nblk, _wr, 0)

      ch = (sw // 16) * 16
      old = cvbuf[pl.ds(ch, 16)]
      cvbuf[pl.ds(ch, 16)] = jnp.where(lanes == (sw % 16), nblk, old)
  pltpu.sync_copy(cvbuf, cnts.at[wid])


_bucket = pl.kernel(
    _bucket_body,
    out_type=[
        jax.ShapeDtypeStruct((NW, 1, E0P // NW), jnp.int32),
        jax.ShapeDtypeStruct((NW, 1, E0P // NW), jnp.int32),
        jax.ShapeDtypeStruct((NW, 8, E1P // NW), jnp.int32),
        jax.ShapeDtypeStruct((NW, 8, E1P // NW), jnp.int32),
        jax.ShapeDtypeStruct((NW, 8, E0P // NW), jnp.int32),
        jax.ShapeDtypeStruct((NW, 8, E0P // NW), jnp.int32),
        jax.ShapeDtypeStruct((NW, 2, BA2P // NW), jnp.int32),
        jax.ShapeDtypeStruct((NW, 2, BA2P // NW), jnp.int32),
        jax.ShapeDtypeStruct((NW, 32), jnp.int32),
    ],
    mesh=_mesh,
    scratch_types=[
        pltpu.VMEM((E1P // NW,), jnp.int32),
        pltpu.VMEM((E1P // NW,), jnp.int32),
        pltpu.VMEM((E1P // NW + G,), jnp.int32),
        pltpu.VMEM((E1P // NW + G,), jnp.int32),
        pltpu.VMEM((32,), jnp.int32),
        pltpu.SemaphoreType.DMA,
    ],
    compiler_params=pltpu.CompilerParams(use_tc_tiling_on_sc=False,
                                         needs_layout_passes=False),
)


def _segsum_body(y0, y1, bg0, bl0, bg1, bl1, bgb1, blb1, bgb2, blb2, cnts,
                 up0p, up1p, ba1p, ba2p,
                 ig0, ig1, lb0, lb1, rows0, rows1, cvbuf,
                 zbuf, win, semi, sg0, sg1):
  cid = lax.axis_index("c")
  sid = lax.axis_index("s")
  wid = sid * 2 + cid
  zv = jnp.zeros((16,), jnp.float32)

  def _zb(i, carry):
    r = i // 4
    c = (i % 4) * 16
    zbuf[r, pl.ds(c, 16)] = zv
    return carry
  lax.fori_loop(0, 256, _zb, 0)

  pltpu.sync_copy(cnts.at[wid], cvbuf)
  cv0 = cvbuf[pl.ds(0, 16)]
  cv1 = cvbuf[pl.ds(16, 16)]

  srcs = (y0, y1)
  bgs = (bg0, bg1, bgb1, bgb2)
  bls = (bl0, bl1, blb1, blb2)
  outs = (up0p, up1p, ba1p, ba2p)
  igs = (ig0, ig1)
  lbs = (lb0, lb1)
  rws = (rows0, rows1)
  sgs = (sg0, sg1)

  stripe = _WR // 16
  for sp in range(4):
    src_id, ep, noutp, nwin = _SPECS[sp]
    src = srcs[src_id]
    bg = bgs[sp]
    bl = bls[sp]
    for w in range(nwin):
      base = w * _WR
      sw = _SWBASE[sp] + w
      nblk = (cv0 if sw < 16 else cv1)[sw % 16]

      for q in range(20):
        pltpu.sync_copy(zbuf, win.at[pl.ds(sid * stripe + q * 64, 64)])
      plsc.subcore_barrier()

      def _stage(j, b):
        ca = pltpu.async_copy(bg.at[wid, w, pl.ds(j * G, G)], igs[b], semi)
        cb = pltpu.async_copy(bl.at[wid, w, pl.ds(j * G, G)], lbs[b], semi)
        ca.wait()
        cb.wait()
        pltpu.async_copy(src.at[igs[b]], rws[b], sgs[b])

      _stage(0, 0)

      def _pair(jj, carry):
        for par in range(2):
          j = jj * 2 + par

          @pl.when(j <= nblk - 2)
          def _():
            _stage(j + 1, 1 - par)

          @pl.when(j <= nblk - 1)
          def _():
            pltpu.make_async_copy(src.at[igs[par]], rws[par],
                                  sgs[par]).wait()
            pltpu.sync_copy(rws[par], win.at[lbs[par]], add=True)
        return carry
      lax.fori_loop(0, (nblk + 1) // 2, _pair, 0)

      plsc.subcore_barrier()
      pltpu.sync_copy(win.at[pl.ds(sid * stripe, stripe)],
                      outs[sp].at[cid, pl.ds(base + sid * stripe, stripe)])


_segsum = pl.kernel(
    _segsum_body,
    out_type=[
        jax.ShapeDtypeStruct((2, N0P, H), jnp.float32),
        jax.ShapeDtypeStruct((2, N1P, H), jnp.float32),
        jax.ShapeDtypeStruct((2, N1P, H), jnp.float32),
        jax.ShapeDtypeStruct((2, N2P, H), jnp.float32),
    ],
    mesh=_mesh,
    scratch_types=[
        pltpu.VMEM((G,), jnp.int32),
        pltpu.VMEM((G,), jnp.int32),
        pltpu.VMEM((G,), jnp.int32),
        pltpu.VMEM((G,), jnp.int32),
        pltpu.VMEM((G, H), jnp.float32),
        pltpu.VMEM((G, H), jnp.float32),
        pltpu.VMEM((32,), jnp.int32),
        pltpu.VMEM((64, H), jnp.float32),
        pltpu.VMEM_SHARED((_WR + 8, H), jnp.float32),
        pltpu.SemaphoreType.DMA,
        pltpu.SemaphoreType.DMA,
        pltpu.SemaphoreType.DMA,
    ],
    compiler_params=pltpu.CompilerParams(use_tc_tiling_on_sc=False,
                                         needs_layout_passes=False),
)


_E0NB = E0P // NW // GP


def _ph_body(vpad, dstp, srcp, b0e, emp,
             gdst, gsrc, rows_a, rows_b, b0v, acc, sem):
  cid = lax.axis_index("c")
  sid = lax.axis_index("s")
  wid = sid * 2 + cid
  zv = jnp.zeros((16,), jnp.float32)

  def _za(i, carry):
    acc[i, pl.ds(0, 16)] = zv
    return carry
  lax.fori_loop(0, 136, _za, 0)

  pltpu.sync_copy(b0e, b0v)
  chunk = E0P // NW
  pltpu.sync_copy(dstp.at[pl.ds(wid * chunk, chunk)], gdst)
  pltpu.sync_copy(srcp.at[pl.ds(wid * chunk, chunk)], gsrc)

  def _blk(j, carry):
    ca = pltpu.async_copy(vpad.at[gsrc.at[pl.ds(j * GP, GP)]], rows_a, sem)
    cb = pltpu.async_copy(vpad.at[gdst.at[pl.ds(j * GP, GP)]], rows_b, sem)
    ca.wait()
    cb.wait()

    def _kk(k, c2):
      dv = gdst[pl.ds(j * GP + k * 16, 16)]
      ebv = plsc.load_gather(b0v, [dv])
      for i in range(16):
        e = k * 16 + i
        m = jnp.maximum(rows_a[e, pl.ds(0, 16)], rows_b[e, pl.ds(0, 16)])
        plsc.addupdate(acc.at[ebv[i]], m)
      return c2
    lax.fori_loop(0, GP // 16, _kk, 0)
    return carry
  lax.fori_loop(0, _E0NB, _blk, 0)
  pltpu.sync_copy(acc.at[pl.ds(0, B)], emp.at[wid])


_ph_sc = pl.kernel(
    _ph_body,
    out_type=[jax.ShapeDtypeStruct((NW, B, 16), jnp.float32)],
    mesh=_mesh,
    scratch_types=[
        pltpu.VMEM((_E0NB * GP,), jnp.int32),
        pltpu.VMEM((_E0NB * GP,), jnp.int32),
        pltpu.VMEM((GP, 16), jnp.float32),
        pltpu.VMEM((GP, 16), jnp.float32),
        pltpu.VMEM((N0 + 8,), jnp.int32),
        pltpu.VMEM((136, 16), jnp.float32),
        pltpu.SemaphoreType.DMA,
    ],
    compiler_params=pltpu.CompilerParams(use_tc_tiling_on_sc=False,
                                         needs_layout_passes=False),
)


def _dot(a, b):
  return jnp.dot(a, b, preferred_element_type=jnp.float32)


def _embed_tc(x, w, b):
  n = x.shape[0]

  def body(x_r, w_r, b_r, o_r):
    o_r[...] = _dot(x_r[...], w_r[...]) + b_r[...]

  return pl.pallas_call(
      body,
      grid=(n // BR,),
      in_specs=[
          pl.BlockSpec((BR, DF), lambda i: (i, 0)),
          pl.BlockSpec((DF, H), lambda i: (0, 0)),
          pl.BlockSpec((1, H), lambda i: (0, 0)),
      ],
      out_specs=pl.BlockSpec((BR, H), lambda i: (i, 0)),
      out_shape=jax.ShapeDtypeStruct((n, H), jnp.float32),
  )(x, w, b.reshape(1, H))


def _relu(x):
  return jnp.maximum(x, 0.0)


def _layer_tc(y, a1, a2, w1, b1, w2, b2, wc, bc):
  n = y.shape[0]
  has1 = a1 is not None
  has2 = a2 is not None

  def body(*refs):
    it = iter(refs)
    y_r = next(it)
    a1_r = next(it) if has1 else None
    a2_r = next(it) if has2 else None
    w1_r, b1_r, w2_r, b2_r, wc_r, bc_r, o_r = [next(it) for _ in range(7)]
    yv = y_r[...]
    u = yv + a1_r[0] + a1_r[1] if has1 else yv
    t1 = _relu(_dot(u, w1_r[...]) + b1_r[...])
    v = yv + a2_r[0] + a2_r[1] if has2 else yv
    t2 = _relu(_dot(v, w2_r[...]) + b2_r[...])
    o_r[...] = _relu(_dot(t1 + t2, wc_r[...]) + bc_r[...])

  specs = [pl.BlockSpec((BR, H), lambda i: (i, 0))]
  args = [y]
  for a in (a1, a2):
    if a is not None:
      specs.append(pl.BlockSpec((2, BR, H), lambda i: (0, i, 0)))
      args.append(a)
  for wgt in (w1, b1.reshape(1, H), w2, b2.reshape(1, H), wc, bc.reshape(1, H)):
    specs.append(pl.BlockSpec(wgt.shape, lambda i: tuple(0 for _ in wgt.shape)))
    args.append(wgt)

  return pl.pallas_call(
      body,
      grid=(n // BR,),
      in_specs=specs,
      out_specs=pl.BlockSpec((BR, H), lambda i: (i, 0)),
      out_shape=jax.ShapeDtypeStruct((n, H), jnp.float32),
  )(*args)


def _final_tc(y, z, a1, a2, w1, b1, w2, b2, wc, bc, batch3, emit_cur):
  n = y.shape[0]
  has1 = a1 is not None
  has2 = a2 is not None

  def body(*refs):
    it = iter(refs)
    y_r = next(it)
    z_r = next(it)
    a1_r = next(it) if has1 else None
    a2_r = next(it) if has2 else None
    w1_r, b1_r, w2_r, b2_r, wc_r, bc_r, bat_r = [next(it) for _ in range(7)]
    if emit_cur:
      cur_r = next(it)
    pool_r = next(it)
    yv = y_r[...]
    u = yv + a1_r[0] + a1_r[1] if has1 else yv
    t1 = _relu(_dot(u, w1_r[...]) + b1_r[...])
    v = yv + a2_r[0] + a2_r[1] if has2 else yv
    t2 = _relu(_dot(v, w2_r[...]) + b2_r[...])
    cur = z_r[...] + _relu(_dot(t1 + t2, wc_r[...]) + bc_r[...])
    if emit_cur:
      cur_r[...] = cur
    i = pl.program_id(0)

    @pl.when(i == 0)
    def _():
      pool_r[...] = jnp.zeros((B, H), jnp.float32)

    bvec = bat_r[0, 0, :]
    oh = (bvec[:, None] == lax.broadcasted_iota(jnp.int32, (BR, B), 1)
          ).astype(jnp.float32)
    pool_r[...] += lax.dot_general(
        oh, cur, dimension_numbers=(((0,), (0,)), ((), ())),
        preferred_element_type=jnp.float32)

  specs = [pl.BlockSpec((BR, H), lambda i: (i, 0)),
           pl.BlockSpec((BR, H), lambda i: (i, 0))]
  args = [y, z]
  for a in (a1, a2):
    if a is not None:
      specs.append(pl.BlockSpec((2, BR, H), lambda i: (0, i, 0)))
      args.append(a)
  for wgt in (w1, b1.reshape(1, H), w2, b2.reshape(1, H), wc, bc.reshape(1, H)):
    specs.append(pl.BlockSpec(wgt.shape, lambda i: tuple(0 for _ in wgt.shape)))
    args.append(wgt)
  specs.append(pl.BlockSpec((1, 1, BR), lambda i: (i, 0, 0)))
  args.append(batch3)

  out_specs = []
  out_shape = []
  if emit_cur:
    out_specs.append(pl.BlockSpec((BR, H), lambda i: (i, 0)))
    out_shape.append(jax.ShapeDtypeStruct((n, H), jnp.float32))
  out_specs.append(pl.BlockSpec((B, H), lambda i: (0, 0)))
  out_shape.append(jax.ShapeDtypeStruct((B, H), jnp.float32))

  return pl.pallas_call(
      body,
      grid=(n // BR,),
      in_specs=specs,
      out_specs=out_specs,
      out_shape=out_shape,
  )(*args)


def _v_tc(x, w1, b1, w2, b2, batch3):
  n = x.shape[0]

  def body(x_r, w1_r, b1_r, w2_r, b2_r, bat_r, vp_r, nm_r):
    h = _relu(_dot(x_r[...], w1_r[...]) + b1_r[...])
    v = jax.nn.sigmoid(_dot(h, w2_r[...]) + b2_r[...])
    vp = jnp.concatenate(
        [v, jnp.ones((BR, 1), jnp.float32), jnp.zeros((BR, 7), jnp.float32)],
        axis=1)
    vp_r[...] = vp
    i = pl.program_id(0)

    @pl.when(i == 0)
    def _():
      nm_r[...] = jnp.zeros((B, 16), jnp.float32)

    bvec = bat_r[0, 0, :]
    oh = (bvec[:, None] == lax.broadcasted_iota(jnp.int32, (BR, B), 1)
          ).astype(jnp.float32)
    nm_r[...] += lax.dot_general(
        oh, vp, dimension_numbers=(((0,), (0,)), ((), ())),
        preferred_element_type=jnp.float32)

  return pl.pallas_call(
      body,
      grid=(n // BR,),
      in_specs=[
          pl.BlockSpec((BR, H), lambda i: (i, 0)),
          pl.BlockSpec((H, FH), lambda i: (0, 0)),
          pl.BlockSpec((1, FH), lambda i: (0, 0)),
          pl.BlockSpec((FH, NF), lambda i: (0, 0)),
          pl.BlockSpec((1, NF), lambda i: (0, 0)),
          pl.BlockSpec((1, 1, BR), lambda i: (i, 0, 0)),
      ],
      out_specs=[
          pl.BlockSpec((BR, 16), lambda i: (i, 0)),
          pl.BlockSpec((B, 16), lambda i: (0, 0)),
      ],
      out_shape=[
          jax.ShapeDtypeStruct((n, 16), jnp.float32),
          jax.ShapeDtypeStruct((B, 16), jnp.float32),
      ],
  )(x, w1, b1.reshape(1, FH), w2, b2.reshape(1, NF), batch3)


def _head_tc(pools, nms, emps, phws, phbs, l1ws, l1bs, l2w, l2b):

  def body(p0_r, p1_r, p2_r, nm0_r, nm1_r, e0_r, e1_r,
           pw0_r, pb0_r, pw1_r, pb1_r,
           lw0_r, lb0_r, lw1_r, lb1_r, lw2_r, lb2_r, l2w_r, l2b_r, o_r):
    def ph(nm_r, emp_r, pw_r, pb_r):
      em_t = jnp.sum(emp_r[...], axis=0)
      ce = jnp.clip(em_t[:, 8:9], 1.0, None)
      emv = em_t[:, 0:8] / ce
      nm_t = nm_r[...]
      c0 = jnp.clip(nm_t[:, 8:9], 1.0, None)
      nmv = nm_t[:, 0:8] / c0
      feat = jnp.concatenate([nmv, emv], axis=1)
      return _relu(_dot(feat, pw_r[...]) + pb_r[...])

    ph0 = ph(nm0_r, e0_r, pw0_r, pb0_r)
    ph1 = ph(nm1_r, e1_r, pw1_r, pb1_r)
    phe = 0.5 * (ph0 + ph1)
    x = (_relu(_dot(p0_r[...], lw0_r[...]) + lb0_r[...]) +
         _relu(_dot(p1_r[...], lw1_r[...]) + lb1_r[...]) +
         _relu(_dot(p2_r[...], lw2_r[...]) + lb2_r[...]))
    o_r[...] = _dot(jnp.concatenate([x, phe], axis=1), l2w_r[...]) + l2b_r[...]

  args = [pools[0], pools[1], pools[2], nms[0], nms[1], emps[0], emps[1],
          phws[0], phbs[0].reshape(1, OPH), phws[1], phbs[1].reshape(1, OPH),
          l1ws[0], l1bs[0].reshape(1, 2 * H), l1ws[1], l1bs[1].reshape(1, 2 * H),
          l1ws[2], l1bs[2].reshape(1, 2 * H), l2w, l2b.reshape(1, 16)]
  return pl.pallas_call(
      body,
      out_shape=jax.ShapeDtypeStruct((B, 16), jnp.float32),
  )(*args)


def kernel(x0, x1, x2, up_index0, up_index1, bdry1_src, bdry1_dst,
           bdry2_src, bdry2_dst, batch0, batch1, batch2, params):
  p = params
  g0, s0 = _pad_idx(up_index0[1], up_index0[0], E0P)
  g1, s1 = _pad_idx(up_index1[1], up_index1[0], E1P)
  gb1, sb1 = _pad_idx(bdry1_src, bdry1_dst, E0P)
  gb2, sb2 = _pad_idx(bdry2_src, bdry2_dst, BA2P)

  buckets = _bucket(g0, s0, g1, s1, gb1, sb1, gb2, sb2)

  z0 = _embed_tc(x0, p["embed_W"][0], p["embed_b"][0])
  z1 = _embed_tc(x1, p["embed_W"][1], p["embed_b"][1])
  z2 = _embed_tc(x2, p["embed_W"][2], p["embed_b"][2])

  batch3 = [batch0.reshape(-1, 1, BR).astype(jnp.int32),
            batch1.reshape(-1, 1, BR).astype(jnp.int32),
            batch2.reshape(-1, 1, BR).astype(jnp.int32)]

  y = [z0, z1, z2]
  zs = [z0, z1, z2]
  pools = [None, None, None]
  cur0 = None
  for l in range(L):
    if l == 0:
      w1l = [p["Wup0"][d, 1:, :] for d in range(3)]
      b1l = [p["bup0"][d] for d in range(3)]
      w2l = [p["Wb0"][d, 1:, :] for d in range(3)]
      b2l = [p["bb0"][d] for d in range(3)]
      wcl = [p["Wc0"][d] for d in range(3)]
      bcl = [p["bc0"][d] for d in range(3)]
    else:
      w1l = [p["WupR"][l - 1, d] for d in range(3)]
      b1l = [p["bupR"][l - 1, d] for d in range(3)]
      w2l = [p["WbR"][l - 1, d] for d in range(3)]
      b2l = [p["bbR"][l - 1, d] for d in range(3)]
      wcl = [p["WcR"][l - 1, d] for d in range(3)]
      bcl = [p["bcR"][l - 1, d] for d in range(3)]

    up0p, up1p, ba1p, ba2p = _segsum(y[0], y[1], *buckets)
    a1s = [up0p, up1p, None]
    a2s = [None, ba1p, ba2p]
    if l < L - 1:
      y = [_layer_tc(y[d], a1s[d], a2s[d], w1l[d], b1l[d], w2l[d], b2l[d],
                     wcl[d], bcl[d]) for d in range(3)]
    else:
      cur0, pools[0] = _final_tc(y[0], zs[0], a1s[0], a2s[0], w1l[0], b1l[0],
                                 w2l[0], b2l[0], wcl[0], bcl[0], batch3[0],
                                 emit_cur=True)
      pools[1], = _final_tc(y[1], zs[1], a1s[1], a2s[1], w1l[1], b1l[1],
                            w2l[1], b2l[1], wcl[1], bcl[1], batch3[1],
                            emit_cur=False)
      pools[2], = _final_tc(y[2], zs[2], a1s[2], a2s[2], w1l[2], b1l[2],
                            w2l[2], b2l[2], wcl[2], bcl[2], batch3[2],
                            emit_cur=False)

  vpad0, nm0 = _v_tc(z0, p["filW1"][0], p["filb1"][0], p["filW2"][0],
                     p["filb2"][0], batch3[0])
  vpad1, nm1 = _v_tc(cur0, p["filW1"][1], p["filb1"][1], p["filW2"][1],
                     p["filb2"][1], batch3[0])

  padrows = jnp.zeros((8, 16), jnp.float32)
  b0e = jnp.concatenate([batch0.astype(jnp.int32),
                         jnp.full((8,), B, jnp.int32)])
  dpad = jnp.full((E0P - E0,), N0, jnp.int32)
  dstp = jnp.concatenate([up_index0[0].astype(jnp.int32), dpad])
  srcp = jnp.concatenate([up_index0[1].astype(jnp.int32), dpad])

  emp0, = _ph_sc(jnp.concatenate([vpad0, padrows]), dstp, srcp, b0e)
  emp1, = _ph_sc(jnp.concatenate([vpad1, padrows]), dstp, srcp, b0e)

  l2w = jnp.concatenate([p["lin2W"], jnp.zeros((2 * H + OPH, 6), jnp.float32)],
                        axis=1)
  l2b = jnp.concatenate([p["lin2b"], jnp.zeros((6,), jnp.float32)])
  out16 = _head_tc(pools, [nm0, nm1], [emp0, emp1],
                   [p["phW"][0], p["phW"][1]], [p["phb"][0], p["phb"][1]],
                   [p["lin1W"][d] for d in range(3)],
                   [p["lin1b"][d] for d in range(3)], l2w, l2b)
  return out16[:, :NC]
